# baseline (speedup 1.0000x reference)
import functools

import jax
import jax.numpy as jnp
from jax import lax
from jax.experimental import pallas as pl
from jax.experimental.pallas import tpu as pltpu
from jax.experimental.pallas import tpu_sc as plsc

B = 16384
N_COUNTRIES = 16
Q = 3
H = 4
D = Q * H

NC = 2
NS = 16
L = 16
NW = NC * NS
BPW = B // NW
GROUPS = BPW // L


def _body(c_hbm, x_hbm, tab_hbm, phi_hbm, out_hbm,
          c_v, x_v, tab_v, phi_v, out_v, sem_c, sem_x, sem_t, sem_p, sem_o):
    wid = lax.axis_index("s") * NC + lax.axis_index("c")
    base = wid * BPW

    cp_c = pltpu.async_copy(c_hbm.at[pl.ds(base, BPW)], c_v, sem_c)
    cp_x = pltpu.async_copy(x_hbm.at[pl.ds(base, BPW)], x_v, sem_x)
    cp_p = pltpu.async_copy(phi_hbm, phi_v, sem_p)
    cp_t = pltpu.async_copy(tab_hbm, tab_v, sem_t)
    lane = lax.iota(jnp.int32, L)
    cp_p.wait()
    phiv = plsc.load_gather(phi_v, [lane * 0])
    cp_t.wait()
    cp_c.wait()
    cp_x.wait()

    def step(t, carry):
        for u in range(2):
            b0 = t * (2 * L) + u * L
            cb = c_v[pl.ds(b0, L)] * jnp.int32(D)
            y = x_v[pl.ds(b0, L)] * phiv
            for p in range(D):
                tv = plsc.load_gather(tab_v, [cb + jnp.int32(p)])
                out_v[pl.ds(p * BPW + b0, L)] = tv + y
        return carry

    lax.fori_loop(0, GROUPS // 2, step, 0)

    cps = [
        pltpu.async_copy(
            out_v.at[pl.ds(p * BPW, BPW)],
            out_hbm.at[p // H, p % H, pl.ds(base, BPW)],
            sem_o,
        )
        for p in range(D)
    ]
    for cp in cps:
        cp.wait()


@jax.jit
def _run(c, xf, tab, phi):
    mesh = plsc.VectorSubcoreMesh(core_axis_name="c", subcore_axis_name="s")
    f = functools.partial(
        pl.kernel,
        out_type=jax.ShapeDtypeStruct((Q, H, B), jnp.float32),
        mesh=mesh,
        scratch_types=[
            pltpu.VMEM((BPW,), jnp.int32),
            pltpu.VMEM((BPW,), jnp.float32),
            pltpu.VMEM((N_COUNTRIES * D,), jnp.float32),
            pltpu.VMEM((1,), jnp.float32),
            pltpu.VMEM((D * BPW,), jnp.float32),
            pltpu.SemaphoreType.DMA,
            pltpu.SemaphoreType.DMA,
            pltpu.SemaphoreType.DMA,
            pltpu.SemaphoreType.DMA,
            pltpu.SemaphoreType.DMA,
        ],
        compiler_params=pltpu.CompilerParams(needs_layout_passes=False),
    )(_body)
    return f(c, xf, tab, phi)


def kernel(x, country_codes, intercept_tensors, phi_tensors):
    c = country_codes.reshape(B).astype(jnp.int32)
    xf = x.reshape(B)
    tab = intercept_tensors.reshape(N_COUNTRIES * D)
    out = _run(c, xf, tab, phi_tensors)
    return jnp.transpose(out, (2, 0, 1))


# speedup vs baseline: 1.0293x; 1.0293x over previous
import functools

import jax
import jax.numpy as jnp
from jax import lax
from jax.experimental import pallas as pl
from jax.experimental.pallas import tpu as pltpu
from jax.experimental.pallas import tpu_sc as plsc

B = 16384
N_COUNTRIES = 16
Q = 3
H = 4
D = Q * H

NC = 2
NS = 16
L = 16
NW = NC * NS
BPW = B // NW
GROUPS = BPW // L


def _body(c_hbm, x_hbm, tab_hbm, phi_hbm, out_hbm,
          c_v, x_v, tab_v, tabt_v, phi_v, out_v, sem_c, sem_x, sem_t, sem_p, sem_o):
    wid = lax.axis_index("s") * NC + lax.axis_index("c")
    base = wid * BPW

    cp_c = pltpu.async_copy(c_hbm.at[pl.ds(base, BPW)], c_v, sem_c)
    cp_x = pltpu.async_copy(x_hbm.at[pl.ds(base, BPW)], x_v, sem_x)
    cp_p = pltpu.async_copy(phi_hbm, phi_v, sem_p)
    cp_t = pltpu.async_copy(tab_hbm, tab_v, sem_t)
    lane = lax.iota(jnp.int32, L)
    cp_p.wait()
    phiv = plsc.load_gather(phi_v, [lane * 0])
    cp_t.wait()
    for p in range(D):
        tabt_v[pl.ds(p * L, L)] = plsc.load_gather(tab_v, [lane * jnp.int32(D) + jnp.int32(p)])
    cp_c.wait()
    cp_x.wait()

    def step(t, carry):
        for u in range(2):
            b0 = t * (2 * L) + u * L
            cb = c_v[pl.ds(b0, L)]
            y = x_v[pl.ds(b0, L)] * phiv
            for p in range(D):
                tv = plsc.load_gather(tabt_v.at[pl.ds(p * L, L)], [cb])
                out_v[pl.ds(p * BPW + b0, L)] = tv + y
        return carry

    lax.fori_loop(0, GROUPS // 2, step, 0)

    cps = [
        pltpu.async_copy(
            out_v.at[pl.ds(p * BPW, BPW)],
            out_hbm.at[p // H, p % H, pl.ds(base, BPW)],
            sem_o,
        )
        for p in range(D)
    ]
    for cp in cps:
        cp.wait()


@jax.jit
def _run(c, xf, tab, phi):
    mesh = plsc.VectorSubcoreMesh(core_axis_name="c", subcore_axis_name="s")
    f = functools.partial(
        pl.kernel,
        out_type=jax.ShapeDtypeStruct((Q, H, B), jnp.float32),
        mesh=mesh,
        scratch_types=[
            pltpu.VMEM((BPW,), jnp.int32),
            pltpu.VMEM((BPW,), jnp.float32),
            pltpu.VMEM((N_COUNTRIES * D,), jnp.float32),
            pltpu.VMEM((D * N_COUNTRIES,), jnp.float32),
            pltpu.VMEM((1,), jnp.float32),
            pltpu.VMEM((D * BPW,), jnp.float32),
            pltpu.SemaphoreType.DMA,
            pltpu.SemaphoreType.DMA,
            pltpu.SemaphoreType.DMA,
            pltpu.SemaphoreType.DMA,
            pltpu.SemaphoreType.DMA,
        ],
        compiler_params=pltpu.CompilerParams(needs_layout_passes=False),
    )(_body)
    return f(c, xf, tab, phi)


def kernel(x, country_codes, intercept_tensors, phi_tensors):
    c = country_codes.reshape(B).astype(jnp.int32)
    xf = x.reshape(B)
    tab = intercept_tensors.reshape(N_COUNTRIES * D)
    out = _run(c, xf, tab, phi_tensors)
    return jnp.transpose(out, (2, 0, 1))
